# baseline (speedup 1.0000x reference)
import functools

import jax
import jax.numpy as jnp
from jax import lax
from jax.experimental import pallas as pl
from jax.experimental.pallas import tpu as pltpu
from jax.experimental.pallas import tpu_sc as plsc

VOCAB = 37
HIDDEN = 128
NC, NS = 2, 16
NW = NC * NS
B = 4096 * 200
CHUNK = 128
NROWS = B // CHUNK
NCHUNK = NROWS // NW
NBUF = 6
LOOKAHEAD = 3


def _body(idx_hbm, table_hbm, out_hbm, idx_v, rows_v, table_sp, gsem, wsem):
    sid = lax.axis_index("s")
    wid = sid * NC + lax.axis_index("c")
    first = wid * NCHUNK

    pltpu.sync_copy(table_hbm, table_sp.at[sid])

    pltpu.sync_copy(idx_hbm.at[pl.ds(first, NCHUNK)], idx_v)
    plsc.subcore_barrier()

    for p in range(LOOKAHEAD):
        pltpu.async_copy(table_sp.at[sid].at[idx_v.at[p]], rows_v.at[p], gsem.at[p])

    def chunk_body(j, carry):
        b = lax.rem(j, NBUF)

        @pl.when(j + LOOKAHEAD < NCHUNK)
        def _():
            nb = lax.rem(j + LOOKAHEAD, NBUF)

            @pl.when(j + LOOKAHEAD >= NBUF)
            def _():
                pltpu.make_async_copy(rows_v.at[nb], out_hbm.at[first], wsem.at[nb]).wait()

            pltpu.async_copy(table_sp.at[sid].at[idx_v.at[j + LOOKAHEAD]], rows_v.at[nb], gsem.at[nb])

        pltpu.make_async_copy(table_sp.at[sid].at[idx_v.at[j]], rows_v.at[b], gsem.at[b]).wait()
        pltpu.async_copy(rows_v.at[b], out_hbm.at[first + j], wsem.at[b])
        return carry

    lax.fori_loop(0, NCHUNK, chunk_body, 0)
    for p in range(NBUF):
        pltpu.make_async_copy(rows_v.at[p], out_hbm.at[first], wsem.at[p]).wait()


@functools.partial(
    pl.kernel,
    out_type=jax.ShapeDtypeStruct((NROWS, CHUNK, HIDDEN), jnp.float32),
    mesh=plsc.VectorSubcoreMesh(core_axis_name="c", subcore_axis_name="s"),
    scratch_types=[
        pltpu.VMEM((NCHUNK, CHUNK), jnp.int32),
        pltpu.VMEM((NBUF, CHUNK, HIDDEN), jnp.float32),
        pltpu.VMEM_SHARED((NS, VOCAB, HIDDEN), jnp.float32),
        pltpu.SemaphoreType.DMA((NBUF,)),
        pltpu.SemaphoreType.DMA((NBUF,)),
    ],
)
def _sc_gather(idx_hbm, table_hbm, out_hbm, idx_v, rows_v, table_sp, gsem, wsem):
    _body(idx_hbm, table_hbm, out_hbm, idx_v, rows_v, table_sp, gsem, wsem)


def kernel(indices, table):
    mask = jnp.ones((VOCAB, 1), dtype=table.dtype).at[0].set(0.0)
    table = table * mask
    idx = indices.reshape(NROWS, CHUNK).astype(jnp.int32)
    out = _sc_gather(idx, table)
    return out.reshape(indices.shape[0], indices.shape[1], HIDDEN)


# speedup vs baseline: 1.0075x; 1.0075x over previous
import functools

import jax
import jax.numpy as jnp
from jax import lax
from jax.experimental import pallas as pl
from jax.experimental.pallas import tpu as pltpu
from jax.experimental.pallas import tpu_sc as plsc

VOCAB = 37
HIDDEN = 128
NC, NS = 2, 16
NW = NC * NS
B = 4096 * 200
CHUNK = 128
NROWS = B // CHUNK
NCHUNK = NROWS // NW
NBUF = 6
LOOKAHEAD = 4


def _body(idx_hbm, table_hbm, out_hbm, idx_v, rows_v, table_sp, gsem, wsem):
    sid = lax.axis_index("s")
    wid = sid * NC + lax.axis_index("c")
    first = wid * NCHUNK

    @pl.when(sid == 0)
    def _():
        pltpu.sync_copy(table_hbm, table_sp)

    pltpu.sync_copy(idx_hbm.at[pl.ds(first, NCHUNK)], idx_v)
    plsc.subcore_barrier()

    for p in range(LOOKAHEAD):
        pltpu.async_copy(table_sp.at[idx_v.at[p]], rows_v.at[p], gsem.at[p])

    def chunk_body(j, carry):
        b = lax.rem(j, NBUF)

        @pl.when(j + LOOKAHEAD < NCHUNK)
        def _():
            nb = lax.rem(j + LOOKAHEAD, NBUF)

            @pl.when(j + LOOKAHEAD >= NBUF)
            def _():
                pltpu.make_async_copy(rows_v.at[nb], out_hbm.at[first], wsem.at[nb]).wait()

            pltpu.async_copy(table_sp.at[idx_v.at[j + LOOKAHEAD]], rows_v.at[nb], gsem.at[nb])

        pltpu.make_async_copy(table_sp.at[idx_v.at[j]], rows_v.at[b], gsem.at[b]).wait()
        pltpu.async_copy(rows_v.at[b], out_hbm.at[first + j], wsem.at[b])
        return carry

    lax.fori_loop(0, NCHUNK, chunk_body, 0)
    for p in range(NBUF):
        pltpu.make_async_copy(rows_v.at[p], out_hbm.at[first], wsem.at[p]).wait()


@functools.partial(
    pl.kernel,
    out_type=jax.ShapeDtypeStruct((NROWS, CHUNK, HIDDEN), jnp.float32),
    mesh=plsc.VectorSubcoreMesh(core_axis_name="c", subcore_axis_name="s"),
    scratch_types=[
        pltpu.VMEM((NCHUNK, CHUNK), jnp.int32),
        pltpu.VMEM((NBUF, CHUNK, HIDDEN), jnp.float32),
        pltpu.VMEM_SHARED((VOCAB, HIDDEN), jnp.float32),
        pltpu.SemaphoreType.DMA((NBUF,)),
        pltpu.SemaphoreType.DMA((NBUF,)),
    ],
)
def _sc_gather(idx_hbm, table_hbm, out_hbm, idx_v, rows_v, table_sp, gsem, wsem):
    _body(idx_hbm, table_hbm, out_hbm, idx_v, rows_v, table_sp, gsem, wsem)


def kernel(indices, table):
    mask = jnp.ones((VOCAB, 1), dtype=table.dtype).at[0].set(0.0)
    table = table * mask
    idx = indices.reshape(NROWS, CHUNK).astype(jnp.int32)
    out = _sc_gather(idx, table)
    return out.reshape(indices.shape[0], indices.shape[1], HIDDEN)
